# baseline (speedup 1.0000x reference)
import functools

import jax
import jax.numpy as jnp
from jax import lax
from jax.experimental import pallas as pl
from jax.experimental.pallas import tpu as pltpu
from jax.experimental.pallas import tpu_sc as plsc

_NC = 2
_NS = 16
_NW = _NC * _NS
_L = 16
_D = 128

_E_TOTAL = 320000
_E_PER_W = _E_TOTAL // _NW
_C = 64
_N_CHUNKS = -(-_E_PER_W // _C)
_E_PAD = _N_CHUNKS * _C
_IDX_PAD = _E_PAD - _E_PER_W


def _compute_chunk(c, srows, drows, obuf):
    lane = lax.iota(jnp.int32, _L)

    def grp_body(g, carry):
        rows = g * _L + lane

        def d_body(d, acc):
            cols = jnp.bitwise_and(lane + d, _D - 1)
            s = plsc.load_gather(srows, [rows, cols])
            t = plsc.load_gather(drows, [rows, cols])
            return acc + s * t

        acc = lax.fori_loop(0, _D, d_body, jnp.zeros((_L,), jnp.float32),
                            unroll=8)
        obuf[pl.ds(c * _C + g * _L, _L)] = acc
        return carry

    lax.fori_loop(0, _C // _L, grp_body, 0)


def _dot_body(table, sidx_hbm, didx_hbm, out_hbm,
              si_a, di_a, si_b, di_b, sr_a, dr_a, sr_b, dr_b, obuf, tbl_sh,
              semi_a, semi_b, semr_a, semr_b):
    sid = lax.axis_index("s")
    wid = sid * _NC + lax.axis_index("c")

    n_nodes = table.shape[0]
    rows_per_sub = (n_nodes // _NS) // 8 * 8
    pltpu.sync_copy(table.at[pl.ds(sid * rows_per_sub, rows_per_sub)],
                    tbl_sh.at[pl.ds(sid * rows_per_sub, rows_per_sub)])
    tail = n_nodes - _NS * rows_per_sub
    if tail:
        @pl.when(sid == _NS - 1)
        def _copy_tail():
            pltpu.sync_copy(table.at[pl.ds(_NS * rows_per_sub, tail)],
                            tbl_sh.at[pl.ds(_NS * rows_per_sub, tail)])

    def fire_idx(c, si, di, sem):
        base = wid * _E_PER_W + c * _C
        pltpu.async_copy(sidx_hbm.at[pl.ds(base, _C)], si, sem)
        pltpu.async_copy(didx_hbm.at[pl.ds(base, _C)], di, sem)

    def wait_idx(si, di, sem):
        pltpu.make_async_copy(sidx_hbm.at[pl.ds(0, _C)], si, sem).wait()
        pltpu.make_async_copy(didx_hbm.at[pl.ds(0, _C)], di, sem).wait()

    def fire_rows(si, di, sr, dr, sem):
        pltpu.async_copy(tbl_sh.at[si], sr, sem)
        pltpu.async_copy(tbl_sh.at[di], dr, sem)

    def wait_rows(si, di, sr, dr, sem):
        pltpu.make_async_copy(tbl_sh.at[si], sr, sem).wait()
        pltpu.make_async_copy(tbl_sh.at[di], dr, sem).wait()

    fire_idx(0, si_a, di_a, semi_a)
    fire_idx(1, si_b, di_b, semi_b)
    plsc.subcore_barrier()
    wait_idx(si_a, di_a, semi_a)
    fire_rows(si_a, di_a, sr_a, dr_a, semr_a)

    def pair_body(i, carry):
        c0 = 2 * i
        c1 = c0 + 1
        wait_idx(si_b, di_b, semi_b)
        fire_rows(si_b, di_b, sr_b, dr_b, semr_b)
        wait_rows(si_a, di_a, sr_a, dr_a, semr_a)
        _compute_chunk(c0, sr_a, dr_a, obuf)
        fire_idx(c0 + 2, si_a, di_a, semi_a)
        wait_rows(si_b, di_b, sr_b, dr_b, semr_b)
        _compute_chunk(c1, sr_b, dr_b, obuf)

        @pl.when(c1 + 2 < _N_CHUNKS)
        def _prefetch_odd():
            fire_idx(c1 + 2, si_b, di_b, semi_b)

        wait_idx(si_a, di_a, semi_a)
        fire_rows(si_a, di_a, sr_a, dr_a, semr_a)
        return carry

    lax.fori_loop(0, (_N_CHUNKS - 1) // 2, pair_body, 0)
    last = _N_CHUNKS - 1
    wait_rows(si_a, di_a, sr_a, dr_a, semr_a)
    _compute_chunk(last, sr_a, dr_a, obuf)

    pltpu.sync_copy(obuf.at[pl.ds(0, _E_PER_W)],
                    out_hbm.at[pl.ds(wid * _E_PER_W, _E_PER_W)])


@jax.jit
def _run(features, src_idx, dst_idx):
    mesh = plsc.VectorSubcoreMesh(core_axis_name="c", subcore_axis_name="s")
    f = functools.partial(
        pl.kernel,
        mesh=mesh,
        compiler_params=pltpu.CompilerParams(needs_layout_passes=False),
        out_type=jax.ShapeDtypeStruct((_E_TOTAL,), jnp.float32),
        scratch_types=[
            pltpu.VMEM((_C,), jnp.int32),
            pltpu.VMEM((_C,), jnp.int32),
            pltpu.VMEM((_C,), jnp.int32),
            pltpu.VMEM((_C,), jnp.int32),
            pltpu.VMEM((_C, _D), jnp.float32),
            pltpu.VMEM((_C, _D), jnp.float32),
            pltpu.VMEM((_C, _D), jnp.float32),
            pltpu.VMEM((_C, _D), jnp.float32),
            pltpu.VMEM((_E_PAD,), jnp.float32),
            pltpu.VMEM_SHARED(features.shape, jnp.float32),
            pltpu.SemaphoreType.DMA,
            pltpu.SemaphoreType.DMA,
            pltpu.SemaphoreType.DMA,
            pltpu.SemaphoreType.DMA,
        ],
    )(_dot_body)
    return f(features, src_idx, dst_idx)


def kernel(features, graph, pos_edge, neg_edge):
    edge = jnp.concatenate([pos_edge, neg_edge], axis=-1)
    edge = jnp.pad(edge, ((0, 0), (0, _IDX_PAD)))
    return _run(features, edge[0], edge[1])
